# baseline (speedup 1.0000x reference)
import functools

import jax
import jax.numpy as jnp
from jax import lax
from jax.experimental import pallas as pl
from jax.experimental.pallas import tpu as pltpu
from jax.experimental.pallas import tpu_sc as plsc

_V = 1000000
_B = 16384
_F = 26
_K = 5
_D = 128
_DU = 16

_NC = 2
_NS = 16
_NW = _NC * _NS
_BPW = _B // _NW
_IPW = _BPW * _F
_GB = 4
_GI = _GB * _F
_NG = _BPW // _GB
_NBUF = 4

_mesh = plsc.VectorSubcoreMesh(core_axis_name="c", subcore_axis_name="s")


@functools.partial(
    pl.kernel,
    out_type=jax.ShapeDtypeStruct((_B, _DU), jnp.float32),
    mesh=_mesh,
    scratch_types=[
        pltpu.VMEM((_IPW,), jnp.int32),
        pltpu.VMEM((_NBUF, _GI, _D), jnp.float32),
        pltpu.VMEM((_BPW, _DU), jnp.float32),
        pltpu.SemaphoreType.DMA((_NBUF,)),
    ],
    compiler_params=pltpu.CompilerParams(use_tc_tiling_on_sc=False),
)
def _sc_gather_sum(tbl_hbm, idx_hbm, acc_hbm, idx_v, buf_v, out_v, sems):
    wid = lax.axis_index("s") * _NC + lax.axis_index("c")
    pltpu.sync_copy(idx_hbm.at[pl.ds(wid * _IPW, _IPW)], idx_v)

    def start(g, d):
        pltpu.async_copy(tbl_hbm.at[idx_v.at[pl.ds(g * _GI, _GI)]],
                         buf_v.at[d], sems.at[d])

    def wait(d):
        pltpu.make_async_copy(
            tbl_hbm.at[pl.ds(0, _GI)], buf_v.at[d], sems.at[d]).wait()

    for d in range(_NBUF):
        start(d, d)

    @pl.loop(0, _NG // _NBUF)
    def outer(t):
        for d in range(_NBUF):
            g = t * _NBUF + d
            wait(d)
            for bb in range(_GB):
                r0 = bb * _F
                v = buf_v[d, r0, pl.ds(0, _DU)]
                for f in range(1, _F):
                    v = v + buf_v[d, r0 + f, pl.ds(0, _DU)]
                out_v[g * _GB + bb, :] = v

            @pl.when(t < _NG // _NBUF - 1)
            def _():
                start(g + _NBUF, d)

    pltpu.sync_copy(out_v, acc_hbm.at[pl.ds(wid * _BPW, _BPW)])


def _combine_body(acc_ref, bias_ref, o_ref):
    s = acc_ref[...]
    clf_l = s[:, :_K]
    m = jnp.max(clf_l, axis=1, keepdims=True)
    e = jnp.exp(clf_l - m)
    clf = e / jnp.sum(e, axis=1, keepdims=True)
    z = s[:, _K:2 * _K] + bias_ref[...]
    lr = 1.0 / (1.0 + jnp.exp(-z))
    o_ref[...] = jnp.sum(clf * lr, axis=1, keepdims=True)


def kernel(x, W_clf, W_lr, bias):
    lrT = W_lr[:, :, 0].T
    tbl = jnp.concatenate(
        [W_clf, lrT, jnp.zeros((_V, _D - 2 * _K), jnp.float32)], axis=1)
    xf = x.reshape(_B * _F)
    acc = _sc_gather_sum(tbl, xf)
    out = pl.pallas_call(
        _combine_body,
        grid=(4,),
        in_specs=[
            pl.BlockSpec((_B // 4, _DU), lambda i: (i, 0)),
            pl.BlockSpec((1, _K), lambda i: (0, 0)),
        ],
        out_specs=pl.BlockSpec((_B // 4, 1), lambda i: (i, 0)),
        out_shape=jax.ShapeDtypeStruct((_B, 1), jnp.float32),
    )(acc, bias.reshape(1, _K))
    return out


# speedup vs baseline: 1.2905x; 1.2905x over previous
import functools

import jax
import jax.numpy as jnp
from jax import lax
from jax.experimental import pallas as pl
from jax.experimental.pallas import tpu as pltpu
from jax.experimental.pallas import tpu_sc as plsc

_V = 1000000
_B = 16384
_F = 26
_K = 5
_D = 16

_NC = 2
_NS = 16
_NW = _NC * _NS
_BPW = _B // _NW
_IPW = _BPW * _F
_GB = 4
_GI = _GB * _F
_NG = _BPW // _GB
_NBUF = 4

_mesh = plsc.VectorSubcoreMesh(core_axis_name="c", subcore_axis_name="s")


@functools.partial(
    pl.kernel,
    out_type=jax.ShapeDtypeStruct((_B, _D), jnp.float32),
    mesh=_mesh,
    scratch_types=[
        pltpu.VMEM((_IPW,), jnp.int32),
        pltpu.VMEM((_NBUF, _GI, _D), jnp.float32),
        pltpu.VMEM((_BPW, _D), jnp.float32),
        pltpu.SemaphoreType.DMA((_NBUF,)),
    ],
    compiler_params=pltpu.CompilerParams(use_tc_tiling_on_sc=False),
)
def _sc_gather_sum(tbl_hbm, idx_hbm, acc_hbm, idx_v, buf_v, out_v, sems):
    wid = lax.axis_index("s") * _NC + lax.axis_index("c")
    pltpu.sync_copy(idx_hbm.at[pl.ds(wid * _IPW, _IPW)], idx_v)

    def start(g, d):
        pltpu.async_copy(tbl_hbm.at[idx_v.at[pl.ds(g * _GI, _GI)]],
                         buf_v.at[d], sems.at[d])

    def wait(d):
        pltpu.make_async_copy(
            tbl_hbm.at[pl.ds(0, _GI)], buf_v.at[d], sems.at[d]).wait()

    for d in range(_NBUF):
        start(d, d)

    @pl.loop(0, _NG // _NBUF)
    def outer(t):
        for d in range(_NBUF):
            g = t * _NBUF + d
            wait(d)
            for bb in range(_GB):
                r0 = bb * _F
                v = buf_v[d, r0, :]
                for f in range(1, _F):
                    v = v + buf_v[d, r0 + f, :]
                out_v[g * _GB + bb, :] = v

            @pl.when(t < _NG // _NBUF - 1)
            def _():
                start(g + _NBUF, d)

    pltpu.sync_copy(out_v, acc_hbm.at[pl.ds(wid * _BPW, _BPW)])


def _combine_body(acc_ref, bias_ref, o_ref):
    s = acc_ref[...]
    clf_l = s[:, :_K]
    m = jnp.max(clf_l, axis=1, keepdims=True)
    e = jnp.exp(clf_l - m)
    clf = e / jnp.sum(e, axis=1, keepdims=True)
    z = s[:, _K:2 * _K] + bias_ref[...]
    lr = 1.0 / (1.0 + jnp.exp(-z))
    o_ref[...] = jnp.sum(clf * lr, axis=1, keepdims=True)


def kernel(x, W_clf, W_lr, bias):
    lrT = W_lr[:, :, 0].T
    tbl = jnp.concatenate(
        [W_clf, lrT, jnp.zeros((_V, _D - 2 * _K), jnp.float32)], axis=1)
    xf = x.reshape(_B * _F)
    acc = _sc_gather_sum(tbl, xf)
    out = pl.pallas_call(
        _combine_body,
        grid=(4,),
        in_specs=[
            pl.BlockSpec((_B // 4, _D), lambda i: (i, 0)),
            pl.BlockSpec((1, _K), lambda i: (0, 0)),
        ],
        out_specs=pl.BlockSpec((_B // 4, 1), lambda i: (i, 0)),
        out_shape=jax.ShapeDtypeStruct((_B, 1), jnp.float32),
    )(acc, bias.reshape(1, _K))
    return out
